# baseline (speedup 1.0000x reference)
import functools

import jax
import jax.numpy as jnp
import numpy as np
from jax import lax
from jax.experimental import pallas as pl
from jax.experimental.pallas import tpu as pltpu
from jax.experimental.pallas import tpu_sc as plsc

N = 10000
E = 320000
D = 128
DH = D // 2
NC = 2
NS = 16
NW = NC * NS
EPW = E // NW
CHUNK = 80
GC = EPW // CHUNK
PACK = 1 << 14
G16 = CHUNK // 16

_PERM = np.concatenate(
    [np.concatenate([32 * j + 2 * np.arange(16),
                     32 * j + 2 * np.arange(16) + 1]) for j in range(4)])


def _sc_spmm(xi32, packed_idx, vals):
  mesh = plsc.VectorSubcoreMesh(
      core_axis_name="c", subcore_axis_name="s", num_cores=NC,
      num_subcores=NS)

  @functools.partial(
      pl.kernel,
      out_type=jax.ShapeDtypeStruct((NC, N, D), jnp.float32),
      mesh=mesh,
      scratch_types=[
          pltpu.VMEM((CHUNK, DH), jnp.int32),
          pltpu.VMEM((16, D), jnp.float32),
          pltpu.VMEM((16, D), jnp.float32),
          pltpu.VMEM((CHUNK,), jnp.int32),
          pltpu.VMEM((CHUNK,), jnp.int32),
          pltpu.VMEM((CHUNK,), jnp.float32),
          pltpu.VMEM((CHUNK,), jnp.float32),
          pltpu.VMEM_SHARED((N, DH), jnp.int32),
          pltpu.VMEM_SHARED((N, D), jnp.float32),
          pltpu.SemaphoreType.DMA,
          pltpu.SemaphoreType.DMA,
          pltpu.SemaphoreType.DMA,
          pltpu.SemaphoreType.DMA,
      ],
      compiler_params=pltpu.CompilerParams(use_tc_tiling_on_sc=False),
  )
  def k(xi_hbm, pidx_hbm, vals_hbm, out_hbm,
        gb, sb0, sb1, pb0, pb1, vb0, vb1, x_sh, acc_sh,
        gsem, ssem, psem, vsem):
    c = lax.axis_index("c")
    s = lax.axis_index("s")
    wid = s * NC + c
    sbufs = (sb0, sb1)
    pbufs = (pb0, pb1)
    vbufs = (vb0, vb1)

    nblk = N // CHUNK

    def _each_x_block(fn):
      for kk in range((nblk + NS - 1) // NS):
        b = s + kk * NS

        @pl.when(b < nblk)
        def _(b=b):
          fn(b * CHUNK)

    def _stage_x(r0):
      pltpu.sync_copy(xi_hbm.at[pl.ds(r0, CHUNK), :], gb)
      pltpu.sync_copy(gb, x_sh.at[pl.ds(r0, CHUNK), :])

    _each_x_block(_stage_x)

    zeros16 = jnp.zeros((16,), jnp.float32)
    for e in range(16):
      for j in range(D // 16):
        sb0[e, pl.ds(j * 16, 16)] = zeros16

    nzb = N // 16

    def _each_zero_block(fn):
      for kk in range((nzb + NS - 1) // NS):
        b = s + kk * NS

        @pl.when(b < nzb)
        def _(b=b):
          fn(b * 16)

    _each_zero_block(
        lambda r0: pltpu.async_copy(
            sb0, acc_sh.at[pl.ds(r0, 16), :], ssem))
    _each_zero_block(
        lambda r0: pltpu.make_async_copy(
            sb0, acc_sh.at[pl.ds(r0, 16), :], ssem).wait())

    plsc.subcore_barrier()

    base = wid * EPW

    def _pidx_start(g, pb):
      pltpu.async_copy(pidx_hbm.at[pl.ds(base + g * CHUNK, CHUNK)],
                       pb, psem)

    def _pidx_wait(pb):
      pltpu.make_async_copy(pidx_hbm.at[pl.ds(base, CHUNK)],
                            pb, psem).wait()

    def _val_start(g, vb):
      pltpu.async_copy(vals_hbm.at[pl.ds(base + g * CHUNK, CHUNK)],
                       vb, vsem)

    def _val_wait(vb):
      pltpu.make_async_copy(vals_hbm.at[pl.ds(base, CHUNK)],
                            vb, vsem).wait()

    def _gather_chunk(pb):
      for t in range(G16):
        pk = pb[pl.ds(t * 16, 16)]
        idx = jnp.bitwise_and(pk, PACK - 1)
        pltpu.async_copy(x_sh.at[idx], gb.at[pl.ds(t * 16, 16), :],
                         gsem)
      for t in range(G16):
        pk = pb[pl.ds(t * 16, 16)]
        idx = jnp.bitwise_and(pk, PACK - 1)
        pltpu.make_async_copy(x_sh.at[idx],
                              gb.at[pl.ds(t * 16, 16), :], gsem).wait()

    def _scale16(t, vb, sb):
      vv = vb[pl.ds(t * 16, 16)]
      for l in range(16):
        e = t * 16 + l
        v = vv[l]
        for j in range(DH // 16):
          pk = gb[e, pl.ds(j * 16, 16)]
          a = lax.bitcast_convert_type(
              jnp.left_shift(pk, 16), jnp.float32)
          b = lax.bitcast_convert_type(
              jnp.bitwise_and(pk, jnp.int32(-65536)), jnp.float32)
          sb[l, pl.ds(j * 32, 16)] = a * v
          sb[l, pl.ds(j * 32 + 16, 16)] = b * v

    def _scatter16(t, pb, sb):
      pk = pb[pl.ds(t * 16, 16)]
      idx = lax.shift_right_logical(pk, 14)
      pltpu.async_copy(sb, acc_sh.at[idx], ssem, add=True)

    def _scatter16_wait(sb):
      izero = jnp.zeros((16,), jnp.int32)
      pltpu.make_async_copy(sb, acc_sh.at[izero], ssem).wait()

    def _chunk(g, gpar, first_two):
      pb = pbufs[gpar]
      vb = vbufs[gpar]
      _pidx_wait(pb)
      _gather_chunk(pb)
      _val_wait(vb)
      for t in range(G16):
        par = (gpar + t) % 2
        if not (first_two and t < 2):
          _scatter16_wait(sbufs[par])
        _scale16(t, vb, sbufs[par])
        _scatter16(t, pb, sbufs[par])

      @pl.when(g + 2 < GC)
      def _():
        _pidx_start(g + 2, pb)
        _val_start(g + 2, vb)

    _pidx_start(0, pb0)
    _val_start(0, vb0)
    _pidx_start(1, pb1)
    _val_start(1, vb1)

    _chunk(0, 0, True)
    _chunk(1, 1, False)

    @pl.loop(2, GC - 1, step=2)
    def _(g0):
      for h in range(2):
        _chunk(g0 + h, h, False)

    pb = pbufs[0]
    vb = vbufs[0]
    _pidx_wait(pb)
    _gather_chunk(pb)
    _val_wait(vb)
    for t in range(G16):
      par = t % 2
      _scatter16_wait(sbufs[par])
      _scale16(t, vb, sbufs[par])
      _scatter16(t, pb, sbufs[par])

    _scatter16_wait(sbufs[1])
    _scatter16_wait(sbufs[0])
    plsc.subcore_barrier()

    _each_x_block(
        lambda r0: pltpu.async_copy(
            acc_sh.at[pl.ds(r0, CHUNK), :],
            out_hbm.at[c, pl.ds(r0, CHUNK), :], gsem))
    _each_x_block(
        lambda r0: pltpu.make_async_copy(
            acc_sh.at[pl.ds(r0, CHUNK), :],
            out_hbm.at[c, pl.ds(r0, CHUNK), :], gsem).wait())

  return k(xi32, packed_idx, vals)


MB = 256


def _tc_body(b1_ref, p_ref, w_ref, relu_ref, z1_ref):
  psum = p_ref[0] + p_ref[1]
  t = jnp.dot(b1_ref[...], psum, preferred_element_type=jnp.float32)
  z1 = jnp.dot(t, w_ref[...], preferred_element_type=jnp.float32)
  z1_ref[...] = z1
  relu_ref[...] = jnp.maximum(z1, 0.0)


def _tc_matmuls(B_1, partials, W_perm):
  nb = B_1.shape[0]
  grid = nb // MB
  return pl.pallas_call(
      _tc_body,
      grid=(grid,),
      in_specs=[
          pl.BlockSpec((MB, N), lambda i: (i, 0)),
          pl.BlockSpec((NC, N, D), lambda i: (0, 0, 0)),
          pl.BlockSpec((D, D), lambda i: (0, 0)),
      ],
      out_specs=[
          pl.BlockSpec((MB, D), lambda i: (i, 0)),
          pl.BlockSpec((MB, D), lambda i: (i, 0)),
      ],
      out_shape=[
          jax.ShapeDtypeStruct((nb, D), jnp.float32),
          jax.ShapeDtypeStruct((nb, D), jnp.float32),
      ],
      compiler_params=pltpu.CompilerParams(
          dimension_semantics=("arbitrary",)),
  )(B_1, partials, W_perm)


def kernel(x, support_indices, support_values, B_1, W):
  xi32 = lax.bitcast_convert_type(
      x.astype(jnp.bfloat16).reshape(N, DH, 2), jnp.int32)
  packed = support_indices[0] * PACK + support_indices[1]
  W_perm = W[jnp.asarray(_PERM), :]
  partials = _sc_spmm(xi32, packed, support_values)
  relu_out, z1 = _tc_matmuls(B_1, partials, W_perm)
  return (relu_out, z1)


# speedup vs baseline: 1.0806x; 1.0806x over previous
import functools

import jax
import jax.numpy as jnp
from jax import lax
from jax.experimental import pallas as pl
from jax.experimental.pallas import tpu as pltpu
from jax.experimental.pallas import tpu_sc as plsc

N = 10000
E = 320000
D = 128
NC = 2
NS = 16
NW = NC * NS
EPW = E // NW
CHUNK = 80
GC = EPW // CHUNK
NBUF = 3
PACK = 1 << 14


def _sc_spmm(x, packed_idx, vals):
  mesh = plsc.VectorSubcoreMesh(
      core_axis_name="c", subcore_axis_name="s", num_cores=NC,
      num_subcores=NS)

  @functools.partial(
      pl.kernel,
      out_type=jax.ShapeDtypeStruct((NC, N, D), jnp.float32),
      mesh=mesh,
      scratch_types=[
          pltpu.VMEM((EPW,), jnp.int32),
          pltpu.VMEM((EPW,), jnp.float32),
          pltpu.VMEM((CHUNK, D), jnp.float32),
          pltpu.VMEM((CHUNK, D), jnp.float32),
          pltpu.VMEM((CHUNK, D), jnp.float32),
          pltpu.VMEM_SHARED((N, D), jnp.float32),
          pltpu.SemaphoreType.DMA,
          pltpu.SemaphoreType.DMA,
      ],
  )
  def k(x_hbm, pidx_hbm, vals_hbm, out_hbm,
        pidx_v, vals_v, buf0, buf1, buf2, acc_sh, gsem, ssem):
    c = lax.axis_index("c")
    s = lax.axis_index("s")
    wid = s * NC + c
    bufs = (buf0, buf1, buf2)

    nblk = N // CHUNK

    def _each_tile_block(fn):
      for kk in range((nblk + NS - 1) // NS):
        b = s + kk * NS

        @pl.when(b < nblk)
        def _(b=b):
          fn(b * CHUNK)

    zeros16 = jnp.zeros((16,), jnp.float32)

    @pl.loop(0, CHUNK)
    def _(e):
      for j in range(D // 16):
        buf0[e, pl.ds(j * 16, 16)] = zeros16

    _each_tile_block(
        lambda r0: pltpu.async_copy(
            buf0, acc_sh.at[pl.ds(r0, CHUNK), :], ssem))
    _each_tile_block(
        lambda r0: pltpu.make_async_copy(
            buf0, acc_sh.at[pl.ds(r0, CHUNK), :], ssem).wait())

    base = wid * EPW
    pltpu.sync_copy(pidx_hbm.at[pl.ds(base, EPW)], pidx_v)
    pltpu.sync_copy(vals_hbm.at[pl.ds(base, EPW)], vals_v)
    plsc.subcore_barrier()

    def _gather_start(g, buf):
      for t in range(CHUNK // 16):
        pk = pidx_v[pl.ds(g * CHUNK + t * 16, 16)]
        idx = jnp.bitwise_and(pk, PACK - 1)
        pltpu.async_copy(x_hbm.at[idx], buf.at[pl.ds(t * 16, 16), :],
                         gsem)

    def _gather_wait(buf):
      for t in range(CHUNK // 16):
        pk = pidx_v[pl.ds(t * 16, 16)]
        idx = jnp.bitwise_and(pk, PACK - 1)
        pltpu.make_async_copy(x_hbm.at[idx],
                              buf.at[pl.ds(t * 16, 16), :], gsem).wait()

    def _scatter_start(g, buf):
      for t in range(CHUNK // 16):
        pk = pidx_v[pl.ds(g * CHUNK + t * 16, 16)]
        idx = lax.shift_right_logical(pk, 14)
        pltpu.async_copy(buf.at[pl.ds(t * 16, 16), :],
                         acc_sh.at[idx], ssem, add=True)

    def _scatter_wait(buf):
      for t in range(CHUNK // 16):
        pk = pidx_v[pl.ds(t * 16, 16)]
        idx = lax.shift_right_logical(pk, 14)
        pltpu.make_async_copy(buf.at[pl.ds(t * 16, 16), :],
                              acc_sh.at[idx], ssem).wait()

    def _scale(g, buf):
      for t in range(CHUNK // 16):
        vv = vals_v[pl.ds(g * CHUNK + t * 16, 16)]
        for l in range(16):
          e = t * 16 + l
          v = vv[l]
          for j in range(D // 16):
            sl = pl.ds(j * 16, 16)
            buf[e, sl] = buf[e, sl] * v

    def _pipe_step(g, bi, wait_scatter, do_gather):
      _gather_wait(bufs[bi])
      if wait_scatter:
        _scatter_wait(bufs[(bi + 1) % NBUF])
      if do_gather:
        _gather_start(g + 1, bufs[(bi + 1) % NBUF])
      _scale(g, bufs[bi])
      _scatter_start(g, bufs[bi])

    _gather_start(0, buf0)
    _gather_wait(buf0)
    _gather_start(1, buf1)
    _scale(0, buf0)
    _scatter_start(0, buf0)
    _pipe_step(1, 1, False, True)

    NGRP = 3
    body_upper = 2 + ((GC - 3) // NGRP) * NGRP

    @pl.loop(2, body_upper, step=NGRP)
    def _(g0):
      for h in range(NGRP):
        _pipe_step(g0 + h, (2 + h) % NBUF, True, True)

    for g in range(body_upper, GC):
      _pipe_step(g, g % NBUF, True, g + 1 < GC)

    _scatter_wait(bufs[(GC - 2) % NBUF])
    _scatter_wait(bufs[(GC - 1) % NBUF])
    plsc.subcore_barrier()

    _each_tile_block(
        lambda r0: pltpu.async_copy(
            acc_sh.at[pl.ds(r0, CHUNK), :],
            out_hbm.at[c, pl.ds(r0, CHUNK), :], gsem))
    _each_tile_block(
        lambda r0: pltpu.make_async_copy(
            acc_sh.at[pl.ds(r0, CHUNK), :],
            out_hbm.at[c, pl.ds(r0, CHUNK), :], gsem).wait())

  return k(x, packed_idx, vals)


MB = 256


def _tc_body(b1_ref, p_ref, w_ref, relu_ref, z1_ref):
  psum = p_ref[0] + p_ref[1]
  t = jnp.dot(b1_ref[...].astype(jnp.bfloat16),
              psum.astype(jnp.bfloat16),
              preferred_element_type=jnp.float32)
  z1 = jnp.dot(t, w_ref[...], preferred_element_type=jnp.float32)
  z1_ref[...] = z1
  relu_ref[...] = jnp.maximum(z1, 0.0)


def _tc_matmuls(B_1, partials, W):
  nb = B_1.shape[0]
  grid = nb // MB
  return pl.pallas_call(
      _tc_body,
      grid=(grid,),
      in_specs=[
          pl.BlockSpec((MB, N), lambda i: (i, 0)),
          pl.BlockSpec((NC, N, D), lambda i: (0, 0, 0)),
          pl.BlockSpec((D, D), lambda i: (0, 0)),
      ],
      out_specs=[
          pl.BlockSpec((MB, D), lambda i: (i, 0)),
          pl.BlockSpec((MB, D), lambda i: (i, 0)),
      ],
      out_shape=[
          jax.ShapeDtypeStruct((nb, D), jnp.float32),
          jax.ShapeDtypeStruct((nb, D), jnp.float32),
      ],
      compiler_params=pltpu.CompilerParams(
          dimension_semantics=("arbitrary",)),
  )(B_1, partials, W)


def kernel(x, support_indices, support_values, B_1, W):
  packed = support_indices[0] * PACK + support_indices[1]
  partials = _sc_spmm(x, packed, support_values)
  relu_out, z1 = _tc_matmuls(B_1, partials, W)
  return (relu_out, z1)
